# Initial kernel scaffold; baseline (speedup 1.0000x reference)
import jax
import jax.numpy as jnp
from jax.experimental import pallas as pl


def kernel(token_ids, weight):
    raise NotImplementedError("write your pallas kernel here")


# speedup vs baseline: 1.5012x; 1.5012x over previous
import functools

import jax
import jax.numpy as jnp
from jax import lax
from jax.experimental import pallas as pl
from jax.experimental.pallas import tpu as pltpu
from jax.experimental.pallas import tpu_sc as plsc

_NC = 2
_NS = 16
_NW = _NC * _NS
_CHUNK = 128
_NBUF = 8


@functools.lru_cache(maxsize=None)
def _build_gather(n_chunks: int, d: int):
    mesh = plsc.VectorSubcoreMesh(core_axis_name="c", subcore_axis_name="s")

    @functools.partial(
        pl.kernel,
        mesh=mesh,
        out_type=jax.ShapeDtypeStruct((_NW, n_chunks, _CHUNK, d), jnp.float32),
        scratch_types=[
            pltpu.VMEM((n_chunks, _CHUNK), jnp.int32),
            pltpu.VMEM((_NBUF, _CHUNK, d), jnp.float32),
            pltpu.SemaphoreType.DMA((_NBUF,)),
        ],
        compiler_params=pltpu.CompilerParams(use_tc_tiling_on_sc=False),
    )
    def gather_kernel(idx_hbm, table_hbm, out_hbm, idx_v, rows_v, sems):
        wid = lax.axis_index("s") * _NC + lax.axis_index("c")
        pltpu.sync_copy(idx_hbm.at[wid], idx_v)

        def gather(t, b):
            return pltpu.make_async_copy(
                table_hbm.at[idx_v.at[t]], rows_v.at[b], sems.at[b])

        for b in range(_NBUF):
            gather(b, b).start()

        def outer(gi, carry):
            g = gi * _NBUF
            for b in range(_NBUF):
                t = g + b
                gather(t, b).wait()
                pltpu.sync_copy(rows_v.at[b], out_hbm.at[wid, t])
                gather(t + _NBUF, b).start()
            return carry

        lax.fori_loop(0, n_chunks // _NBUF - 1, outer, 0)

        for b in range(_NBUF):
            t = n_chunks - _NBUF + b
            gather(t, b).wait()
            pltpu.sync_copy(rows_v.at[b], out_hbm.at[wid, t])

    return gather_kernel


def kernel(token_ids, weight):
    bsz, seq = token_ids.shape
    d = weight.shape[1]
    n = bsz * seq
    assert n % (_NW * _CHUNK * _NBUF) == 0
    n_chunks = n // (_NW * _CHUNK)
    idx = token_ids.reshape(_NW, n_chunks, _CHUNK).astype(jnp.int32)
    out = _build_gather(n_chunks, d)(idx, weight)
    return out.reshape(bsz, seq, d)
